# baseline (speedup 1.0000x reference)
import functools

import jax
import jax.numpy as jnp
import numpy as np
from jax import lax
from jax.experimental import pallas as pl
from jax.experimental.pallas import tpu as pltpu

N = 10000; E = 160000; F = 128; FE = 16; P = 8
D = 256; H = 8; DH = 32; L = 2; NL = 21; AA = 32; NC = 10

RBN = 1000
RBE = 2000
INV_SQRT_DH = 1.0 / np.sqrt(DH)


def _ln(x, g, b):
    mu = jnp.mean(x, -1, keepdims=True)
    v = jnp.mean((x - mu) ** 2, -1, keepdims=True)
    return (x - mu) / jnp.sqrt(v + 1e-5) * g + b


def _node_init_body(lab_ref, nf_ref, lap_ref, Maa_ref, Wn2_ref, Wlap_ref,
                    b_ref, out_ref):
    lab = lab_ref[0, 0, :]
    oh = (lab[:, None] ==
          lax.broadcasted_iota(jnp.int32, (RBN, AA), 1)).astype(jnp.float32)
    h = oh @ Maa_ref[...]
    h += jnp.dot(nf_ref[...], Wn2_ref[...], preferred_element_type=jnp.float32)
    h += jnp.dot(lap_ref[...], Wlap_ref[...], preferred_element_type=jnp.float32)
    out_ref[...] = h + b_ref[...]


def _edge_init_body(ef_ref, We_ref, be_ref, Wpe_ref, e0_ref, pe_ref):
    e0 = jnp.dot(ef_ref[...], We_ref[...],
                 preferred_element_type=jnp.float32) + be_ref[...]
    e0_ref[...] = e0
    pe = jnp.dot(e0, Wpe_ref[...], preferred_element_type=jnp.float32)
    pe_ref[0] = pe[:, :128]
    pe_ref[1] = pe[:, 128:]


def _qkv_body(h_ref, Wq_ref, Wk_ref, Wv_ref, q_ref, k_ref, v_ref):
    h = h_ref[...]
    for W, o in ((Wq_ref, q_ref), (Wk_ref, k_ref), (Wv_ref, v_ref)):
        r = jnp.dot(h, W[...], preferred_element_type=jnp.float32)
        o[0] = r[:, :128]
        o[1] = r[:, 128:]


def _h_update_body(h_ref, ha_ref, WO_ref, bO_ref, g1_ref, b1_ref,
                   W1_ref, bf1_ref, W2_ref, bf2_ref, g2_ref, b2_ref, out_ref):
    h2 = (jnp.dot(ha_ref[0], WO_ref[:128], preferred_element_type=jnp.float32)
          + jnp.dot(ha_ref[1], WO_ref[128:], preferred_element_type=jnp.float32)
          + bO_ref[...] + h_ref[...])
    h2 = _ln(h2, g1_ref[...], b1_ref[...])
    hf = jnp.maximum(
        jnp.dot(h2, W1_ref[...], preferred_element_type=jnp.float32)
        + bf1_ref[...], 0.0)
    hf = jnp.dot(hf, W2_ref[...], preferred_element_type=jnp.float32) + bf2_ref[...]
    out_ref[...] = _ln(h2 + hf, g2_ref[...], b2_ref[...])


def _e_update_body(e_ref, eo_ref, WO_ref, bO_ref, g1_ref, b1_ref,
                   W1_ref, bf1_ref, W2_ref, bf2_ref, g2_ref, b2_ref,
                   Wpe_ref, pe_ref):
    e2 = (jnp.dot(eo_ref[0], WO_ref[:128], preferred_element_type=jnp.float32)
          + jnp.dot(eo_ref[1], WO_ref[128:], preferred_element_type=jnp.float32)
          + bO_ref[...] + e_ref[...])
    e2 = _ln(e2, g1_ref[...], b1_ref[...])
    ef = jnp.maximum(
        jnp.dot(e2, W1_ref[...], preferred_element_type=jnp.float32)
        + bf1_ref[...], 0.0)
    ef = jnp.dot(ef, W2_ref[...], preferred_element_type=jnp.float32) + bf2_ref[...]
    e3 = _ln(e2 + ef, g2_ref[...], b2_ref[...])
    pe = jnp.dot(e3, Wpe_ref[...], preferred_element_type=jnp.float32)
    pe_ref[0] = pe[:, :128]
    pe_ref[1] = pe[:, 128:]


def _readout_body(h_ref, R0W_ref, R0b_ref, R1W_ref, R1b_ref, R2W_ref, R2b_ref,
                  out_ref, acc_ref):
    i = pl.program_id(0)

    @pl.when(i == 0)
    def _():
        acc_ref[...] = jnp.zeros_like(acc_ref)

    acc_ref[...] += jnp.sum(h_ref[...].reshape(RBN // 8, 8, D), axis=0)

    @pl.when(i == pl.num_programs(0) - 1)
    def _():
        hg = (jnp.sum(acc_ref[...], axis=0, keepdims=True) / N)
        y = jnp.maximum(
            jnp.dot(hg, R0W_ref[...], preferred_element_type=jnp.float32)
            + R0b_ref[...], 0.0)
        y = jnp.maximum(
            jnp.dot(y, R1W_ref[...], preferred_element_type=jnp.float32)
            + R1b_ref[...], 0.0)
        out_ref[...] = (jnp.dot(y, R2W_ref[...],
                                preferred_element_type=jnp.float32)
                        + R2b_ref[...])


def _full(shape):
    return pl.BlockSpec(shape, lambda i: tuple(0 for _ in shape))


def _node_init(labels, node_feat, lap, Maa, Wn2, Wlap, b):
    grid = (N // RBN,)
    lab3 = labels.astype(jnp.int32).reshape(N // RBN, 1, RBN)
    return pl.pallas_call(
        _node_init_body,
        grid=grid,
        in_specs=[
            pl.BlockSpec((1, 1, RBN), lambda i: (i, 0, 0)),
            pl.BlockSpec((RBN, F), lambda i: (i, 0)),
            pl.BlockSpec((RBN, P), lambda i: (i, 0)),
            _full((AA, D)), _full((F, D)), _full((P, D)), _full((1, D)),
        ],
        out_specs=pl.BlockSpec((RBN, D), lambda i: (i, 0)),
        out_shape=jax.ShapeDtypeStruct((N, D), jnp.float32),
    )(lab3, node_feat, lap, Maa, Wn2, Wlap, b)


def _edge_init(edge_feat, We, be, Wpe):
    grid = (E // RBE,)
    return pl.pallas_call(
        _edge_init_body,
        grid=grid,
        in_specs=[
            pl.BlockSpec((RBE, FE), lambda i: (i, 0)),
            _full((FE, D)), _full((1, D)), _full((D, D)),
        ],
        out_specs=[
            pl.BlockSpec((RBE, D), lambda i: (i, 0)),
            pl.BlockSpec((2, RBE, 128), lambda i: (0, i, 0)),
        ],
        out_shape=[
            jax.ShapeDtypeStruct((E, D), jnp.float32),
            jax.ShapeDtypeStruct((2, E, 128), jnp.float32),
        ],
    )(edge_feat, We, be, Wpe)


def _qkv(h, Wq, Wk, Wv):
    grid = (N // RBN,)
    os = pl.BlockSpec((2, RBN, 128), lambda i: (0, i, 0))
    sh = jax.ShapeDtypeStruct((2, N, 128), jnp.float32)
    return pl.pallas_call(
        _qkv_body,
        grid=grid,
        in_specs=[pl.BlockSpec((RBN, D), lambda i: (i, 0)),
                  _full((D, D)), _full((D, D)), _full((D, D))],
        out_specs=[os, os, os],
        out_shape=[sh, sh, sh],
    )(h, Wq, Wk, Wv)


def _h_update(h, ha, WO, bO, g1, b1, W1, bf1, W2, bf2, g2, b2):
    grid = (N // RBN,)
    return pl.pallas_call(
        _h_update_body,
        grid=grid,
        in_specs=[
            pl.BlockSpec((RBN, D), lambda i: (i, 0)),
            pl.BlockSpec((2, RBN, 128), lambda i: (0, i, 0)),
            _full((D, D)), _full((1, D)), _full((1, D)), _full((1, D)),
            _full((D, 2 * D)), _full((1, 2 * D)), _full((2 * D, D)),
            _full((1, D)), _full((1, D)), _full((1, D)),
        ],
        out_specs=pl.BlockSpec((RBN, D), lambda i: (i, 0)),
        out_shape=jax.ShapeDtypeStruct((N, D), jnp.float32),
    )(h, ha, WO, bO, g1, b1, W1, bf1, W2, bf2, g2, b2)


def _e_update(e, eo, WO, bO, g1, b1, W1, bf1, W2, bf2, g2, b2, Wpe2):
    grid = (E // RBE,)
    return pl.pallas_call(
        _e_update_body,
        grid=grid,
        in_specs=[
            pl.BlockSpec((RBE, D), lambda i: (i, 0)),
            pl.BlockSpec((2, RBE, 128), lambda i: (0, i, 0)),
            _full((D, D)), _full((1, D)), _full((1, D)), _full((1, D)),
            _full((D, 2 * D)), _full((1, 2 * D)), _full((2 * D, D)),
            _full((1, D)), _full((1, D)), _full((1, D)), _full((D, D)),
        ],
        out_specs=pl.BlockSpec((2, RBE, 128), lambda i: (0, i, 0)),
        out_shape=jax.ShapeDtypeStruct((2, E, 128), jnp.float32),
    )(e, eo, WO, bO, g1, b1, W1, bf1, W2, bf2, g2, b2, Wpe2)


def _readout(h, R0W, R0b, R1W, R1b, R2Wp, R2bp):
    grid = (N // RBN,)
    y = pl.pallas_call(
        _readout_body,
        grid=grid,
        in_specs=[
            pl.BlockSpec((RBN, D), lambda i: (i, 0)),
            _full((D, D // 2)), _full((1, D // 2)),
            _full((D // 2, D // 4)), _full((1, D // 4)),
            _full((D // 4, 16)), _full((1, 16)),
        ],
        out_specs=pl.BlockSpec((1, 16), lambda i: (0, 0)),
        out_shape=jax.ShapeDtypeStruct((1, 16), jnp.float32),
        scratch_shapes=[pltpu.VMEM((8, D), jnp.float32)],
    )(h, R0W, R0b, R1W, R1b, R2Wp, R2bp)
    return y[0, :NC]


def _edge_attention(K2, Q2, V2, pe2, src, dst, need_eout):
    Kh = jnp.concatenate([K2[0], K2[1]], 1).reshape(N, H, DH)
    Qh = jnp.concatenate([Q2[0], Q2[1]], 1).reshape(N, H, DH)
    Vh = jnp.concatenate([V2[0], V2[1]], 1).reshape(N, H, DH)
    pe = jnp.concatenate([pe2[0], pe2[1]], 1).reshape(E, H, DH)
    score = Kh[src] * Qh[dst] * INV_SQRT_DH * pe
    sc = jnp.exp(jnp.clip(jnp.sum(score, -1, keepdims=True), -5.0, 5.0))
    wV = jax.ops.segment_sum(Vh[src] * sc, dst, num_segments=N)
    z = jax.ops.segment_sum(sc, dst, num_segments=N)
    ha = (wV / (z + 1e-6)).reshape(N, D)
    ha2 = jnp.stack([ha[:, :128], ha[:, 128:]], 0)
    if need_eout:
        eo = score.reshape(E, D)
        eo2 = jnp.stack([eo[:, :128], eo[:, 128:]], 0)
    else:
        eo2 = None
    return eo2, ha2


def kernel(node_feat, edge_feat, lap_pos_enc, node_labels, edge_index, params):
    p = params
    src = edge_index[0].astype(jnp.int32)
    dst = edge_index[1].astype(jnp.int32)

    Maa = jnp.pad(p['aa_emb'] @ p['Wn'][:AA], ((0, AA - NL), (0, 0)))
    b0 = (p['bn'] + p['blap']).reshape(1, D)

    h = _node_init(node_labels, node_feat, lap_pos_enc,
                   Maa, p['Wn'][AA:], p['Wlap'], b0)
    e0, pe1 = _edge_init(edge_feat, p['We'], p['be'].reshape(1, D), p['Wpe'][0])

    pe_l = pe1
    for l in range(L):
        Q2, K2, V2 = _qkv(h, p['Wq'][l], p['Wk'][l], p['Wv'][l])
        eo2, ha2 = _edge_attention(K2, Q2, V2, pe_l, src, dst,
                                   need_eout=(l == 0))
        h = _h_update(h, ha2, p['WOh'][l], p['bOh'][l].reshape(1, D),
                      p['ln1h_g'][l].reshape(1, D), p['ln1h_b'][l].reshape(1, D),
                      p['Wf1h'][l], p['bf1h'][l].reshape(1, 2 * D),
                      p['Wf2h'][l], p['bf2h'][l].reshape(1, D),
                      p['ln2h_g'][l].reshape(1, D), p['ln2h_b'][l].reshape(1, D))
        if l == 0:
            pe_l = _e_update(e0, eo2, p['WOe'][l], p['bOe'][l].reshape(1, D),
                             p['ln1e_g'][l].reshape(1, D), p['ln1e_b'][l].reshape(1, D),
                             p['Wf1e'][l], p['bf1e'][l].reshape(1, 2 * D),
                             p['Wf2e'][l], p['bf2e'][l].reshape(1, D),
                             p['ln2e_g'][l].reshape(1, D), p['ln2e_b'][l].reshape(1, D),
                             p['Wpe'][1])

    R2Wp = jnp.pad(p['R2W'], ((0, 0), (0, 16 - NC)))
    R2bp = jnp.pad(p['R2b'], (0, 16 - NC)).reshape(1, 16)
    return _readout(h, p['R0W'], p['R0b'].reshape(1, D // 2),
                    p['R1W'], p['R1b'].reshape(1, D // 4), R2Wp, R2bp)


# speedup vs baseline: 10.3882x; 10.3882x over previous
import dataclasses
import functools

import jax
import jax.numpy as jnp
import numpy as np
from jax import lax
from jax.experimental import pallas as pl
from jax.experimental.pallas import tpu as pltpu
from jax.experimental.pallas import tpu_sc as plsc

N = 10000; E = 160000; F = 128; FE = 16; P = 8
D = 256; H = 8; DH = 32; L = 2; NL = 21; AA = 32; NC = 10

RBN = 1000
RBE = 2000
INV_SQRT_DH = 1.0 / np.sqrt(DH)


def _ln(x, g, b):
    mu = jnp.mean(x, -1, keepdims=True)
    v = jnp.mean((x - mu) ** 2, -1, keepdims=True)
    return (x - mu) / jnp.sqrt(v + 1e-5) * g + b


def _node_init_body(lab_ref, nf_ref, lap_ref, Maa_ref, Wn2_ref, Wlap_ref,
                    b_ref, out_ref):
    lab = lab_ref[0, 0, :]
    oh = (lab[:, None] ==
          lax.broadcasted_iota(jnp.int32, (RBN, AA), 1)).astype(jnp.float32)
    h = oh @ Maa_ref[...]
    h += jnp.dot(nf_ref[...], Wn2_ref[...], preferred_element_type=jnp.float32)
    h += jnp.dot(lap_ref[...], Wlap_ref[...], preferred_element_type=jnp.float32)
    out_ref[...] = h + b_ref[...]


def _edge_init_body(ef_ref, We_ref, be_ref, Wpe_ref, e0_ref, pe_ref):
    e0 = jnp.dot(ef_ref[...], We_ref[...],
                 preferred_element_type=jnp.float32) + be_ref[...]
    e0_ref[...] = e0
    pe = jnp.dot(e0, Wpe_ref[...], preferred_element_type=jnp.float32)
    pe_ref[0] = pe[:, :128]
    pe_ref[1] = pe[:, 128:]


def _qkv_body(h_ref, Wq_ref, Wk_ref, Wv_ref, q_ref, k_ref, v_ref):
    h = h_ref[...]
    for W, o in ((Wq_ref, q_ref), (Wk_ref, k_ref), (Wv_ref, v_ref)):
        r = jnp.dot(h, W[...], preferred_element_type=jnp.float32)
        o[0] = r[:, :128]
        o[1] = r[:, 128:]


def _h_update_body(h_ref, ha_ref, WO_ref, bO_ref, g1_ref, b1_ref,
                   W1_ref, bf1_ref, W2_ref, bf2_ref, g2_ref, b2_ref, out_ref):
    h2 = (jnp.dot(ha_ref[0], WO_ref[:128], preferred_element_type=jnp.float32)
          + jnp.dot(ha_ref[1], WO_ref[128:], preferred_element_type=jnp.float32)
          + bO_ref[...] + h_ref[...])
    h2 = _ln(h2, g1_ref[...], b1_ref[...])
    hf = jnp.maximum(
        jnp.dot(h2, W1_ref[...], preferred_element_type=jnp.float32)
        + bf1_ref[...], 0.0)
    hf = jnp.dot(hf, W2_ref[...], preferred_element_type=jnp.float32) + bf2_ref[...]
    out_ref[...] = _ln(h2 + hf, g2_ref[...], b2_ref[...])


def _e_update_body(e_ref, eo_ref, WO_ref, bO_ref, g1_ref, b1_ref,
                   W1_ref, bf1_ref, W2_ref, bf2_ref, g2_ref, b2_ref,
                   Wpe_ref, pe_ref):
    e2 = (jnp.dot(eo_ref[0], WO_ref[:128], preferred_element_type=jnp.float32)
          + jnp.dot(eo_ref[1], WO_ref[128:], preferred_element_type=jnp.float32)
          + bO_ref[...] + e_ref[...])
    e2 = _ln(e2, g1_ref[...], b1_ref[...])
    ef = jnp.maximum(
        jnp.dot(e2, W1_ref[...], preferred_element_type=jnp.float32)
        + bf1_ref[...], 0.0)
    ef = jnp.dot(ef, W2_ref[...], preferred_element_type=jnp.float32) + bf2_ref[...]
    e3 = _ln(e2 + ef, g2_ref[...], b2_ref[...])
    pe = jnp.dot(e3, Wpe_ref[...], preferred_element_type=jnp.float32)
    pe_ref[0] = pe[:, :128]
    pe_ref[1] = pe[:, 128:]


def _readout_body(h_ref, R0W_ref, R0b_ref, R1W_ref, R1b_ref, R2W_ref, R2b_ref,
                  out_ref, acc_ref):
    i = pl.program_id(0)

    @pl.when(i == 0)
    def _():
        acc_ref[...] = jnp.zeros_like(acc_ref)

    acc_ref[...] += jnp.sum(h_ref[...].reshape(RBN // 8, 8, D), axis=0)

    @pl.when(i == pl.num_programs(0) - 1)
    def _():
        hg = (jnp.sum(acc_ref[...], axis=0, keepdims=True) / N)
        y = jnp.maximum(
            jnp.dot(hg, R0W_ref[...], preferred_element_type=jnp.float32)
            + R0b_ref[...], 0.0)
        y = jnp.maximum(
            jnp.dot(y, R1W_ref[...], preferred_element_type=jnp.float32)
            + R1b_ref[...], 0.0)
        out_ref[...] = (jnp.dot(y, R2W_ref[...],
                                preferred_element_type=jnp.float32)
                        + R2b_ref[...])


def _full(shape):
    return pl.BlockSpec(shape, lambda i: tuple(0 for _ in shape))


def _node_init(labels, node_feat, lap, Maa, Wn2, Wlap, b):
    grid = (N // RBN,)
    lab3 = labels.astype(jnp.int32).reshape(N // RBN, 1, RBN)
    return pl.pallas_call(
        _node_init_body,
        grid=grid,
        in_specs=[
            pl.BlockSpec((1, 1, RBN), lambda i: (i, 0, 0)),
            pl.BlockSpec((RBN, F), lambda i: (i, 0)),
            pl.BlockSpec((RBN, P), lambda i: (i, 0)),
            _full((AA, D)), _full((F, D)), _full((P, D)), _full((1, D)),
        ],
        out_specs=pl.BlockSpec((RBN, D), lambda i: (i, 0)),
        out_shape=jax.ShapeDtypeStruct((N, D), jnp.float32),
    )(lab3, node_feat, lap, Maa, Wn2, Wlap, b)


def _edge_init(edge_feat, We, be, Wpe):
    grid = (E // RBE,)
    return pl.pallas_call(
        _edge_init_body,
        grid=grid,
        in_specs=[
            pl.BlockSpec((RBE, FE), lambda i: (i, 0)),
            _full((FE, D)), _full((1, D)), _full((D, D)),
        ],
        out_specs=[
            pl.BlockSpec((RBE, D), lambda i: (i, 0)),
            pl.BlockSpec((2, RBE, 128), lambda i: (0, i, 0)),
        ],
        out_shape=[
            jax.ShapeDtypeStruct((E, D), jnp.float32),
            jax.ShapeDtypeStruct((2, E, 128), jnp.float32),
        ],
    )(edge_feat, We, be, Wpe)


def _qkv(h, Wq, Wk, Wv):
    grid = (N // RBN,)
    os = pl.BlockSpec((2, RBN, 128), lambda i: (0, i, 0))
    sh = jax.ShapeDtypeStruct((2, N, 128), jnp.float32)
    return pl.pallas_call(
        _qkv_body,
        grid=grid,
        in_specs=[pl.BlockSpec((RBN, D), lambda i: (i, 0)),
                  _full((D, D)), _full((D, D)), _full((D, D))],
        out_specs=[os, os, os],
        out_shape=[sh, sh, sh],
    )(h, Wq, Wk, Wv)


def _h_update(h, ha, WO, bO, g1, b1, W1, bf1, W2, bf2, g2, b2):
    grid = (N // RBN,)
    return pl.pallas_call(
        _h_update_body,
        grid=grid,
        in_specs=[
            pl.BlockSpec((RBN, D), lambda i: (i, 0)),
            pl.BlockSpec((2, RBN, 128), lambda i: (0, i, 0)),
            _full((D, D)), _full((1, D)), _full((1, D)), _full((1, D)),
            _full((D, 2 * D)), _full((1, 2 * D)), _full((2 * D, D)),
            _full((1, D)), _full((1, D)), _full((1, D)),
        ],
        out_specs=pl.BlockSpec((RBN, D), lambda i: (i, 0)),
        out_shape=jax.ShapeDtypeStruct((N, D), jnp.float32),
    )(h, ha, WO, bO, g1, b1, W1, bf1, W2, bf2, g2, b2)


def _e_update(e, eo, WO, bO, g1, b1, W1, bf1, W2, bf2, g2, b2, Wpe2):
    grid = (E // RBE,)
    return pl.pallas_call(
        _e_update_body,
        grid=grid,
        in_specs=[
            pl.BlockSpec((RBE, D), lambda i: (i, 0)),
            pl.BlockSpec((2, RBE, 128), lambda i: (0, i, 0)),
            _full((D, D)), _full((1, D)), _full((1, D)), _full((1, D)),
            _full((D, 2 * D)), _full((1, 2 * D)), _full((2 * D, D)),
            _full((1, D)), _full((1, D)), _full((1, D)), _full((D, D)),
        ],
        out_specs=pl.BlockSpec((2, RBE, 128), lambda i: (0, i, 0)),
        out_shape=jax.ShapeDtypeStruct((2, E, 128), jnp.float32),
    )(e, eo, WO, bO, g1, b1, W1, bf1, W2, bf2, g2, b2, Wpe2)


def _readout(h, R0W, R0b, R1W, R1b, R2Wp, R2bp):
    grid = (N // RBN,)
    y = pl.pallas_call(
        _readout_body,
        grid=grid,
        in_specs=[
            pl.BlockSpec((RBN, D), lambda i: (i, 0)),
            _full((D, D // 2)), _full((1, D // 2)),
            _full((D // 2, D // 4)), _full((1, D // 4)),
            _full((D // 4, 16)), _full((1, 16)),
        ],
        out_specs=pl.BlockSpec((1, 16), lambda i: (0, 0)),
        out_shape=jax.ShapeDtypeStruct((1, 16), jnp.float32),
        scratch_shapes=[pltpu.VMEM((8, D), jnp.float32)],
    )(h, R0W, R0b, R1W, R1b, R2Wp, R2bp)
    return y[0, :NC]


CE = 80
TPE = E // 16
NCH = TPE // CE
NPASS = 3
RPP = 3360
DUMMY = RPP
ACC_R = 3440
ZACC_R = 480
Z_DUMMY = 424
DIVC = 80
ZC = ACC_R // DIVC + ZACC_R // DIVC
ZC_PT = -(-ZC // 16)
NHH = H // 2


def _make_sc_attention():
    mesh = plsc.VectorSubcoreMesh(core_axis_name="c", subcore_axis_name="s")
    out_type = [
        jax.ShapeDtypeStruct((2 * E, 128), jnp.float32),
        jax.ShapeDtypeStruct((2 * N, 128), jnp.float32),
        jax.ShapeDtypeStruct((2 * E, 128), jnp.float32),
        jax.ShapeDtypeStruct((2 * E, 128), jnp.float32),
    ]
    scratch_types = [
        pltpu.VMEM((CE,), jnp.int32),
        pltpu.VMEM((CE,), jnp.int32),
        pltpu.VMEM((CE,), jnp.int32),
        pltpu.VMEM((CE,), jnp.int32),
        pltpu.VMEM((CE,), jnp.int32),
        pltpu.VMEM((CE + 16,), jnp.int32),
        pltpu.VMEM((CE, 128), jnp.float32),
        pltpu.VMEM((CE, 128), jnp.float32),
        pltpu.VMEM((CE, 128), jnp.float32),
        pltpu.VMEM((CE, 128), jnp.float32),
        pltpu.VMEM((CE, 128), jnp.float32),
        pltpu.VMEM((CE, 128), jnp.float32),
        pltpu.VMEM((CE, 128), jnp.float32),
        pltpu.VMEM((DIVC, 128), jnp.float32),
        pltpu.VMEM((16, 128), jnp.float32),
        pltpu.VMEM_SHARED((ACC_R, 128), jnp.float32),
        pltpu.VMEM_SHARED((ZACC_R, 128), jnp.float32),
        pltpu.SemaphoreType.DMA,
        pltpu.SemaphoreType.DMA,
        pltpu.SemaphoreType.DMA,
        pltpu.SemaphoreType.DMA,
    ]

    cp = pltpu.CompilerParams()
    if "needs_layout_passes" in pltpu.CompilerParams.__dataclass_fields__:
        cp = dataclasses.replace(cp, needs_layout_passes=False)

    @functools.partial(pl.kernel, mesh=mesh, out_type=out_type,
                       scratch_types=scratch_types, compiler_params=cp)
    def body(k_hbm, q_hbm, v_hbm, pe_hbm, src_hbm, dst_hbm,
             eo_hbm, ha_hbm, wvs_hbm, zs_hbm,
             src_v, dst_v, srco_v, dst2_v, zdst_v, slot_v,
             kb, qb, vb, peb, eob, wvb, zb, bigb, zdivb,
             wv_sh, z_sh, sem0, sem1, sem2, sem3):
        c = lax.axis_index("c")
        s = lax.axis_index("s")
        coff_n = c * N
        coff_e = c * E
        io = lax.iota(jnp.int32, 16)
        zero16 = jnp.zeros((16,), jnp.float32)
        masks = [io == hh for hh in range(NHH)]
        mlt4 = io < NHH
        ebase = s * TPE

        def zero_phase():
            @pl.loop(0, DIVC)
            def _zero_fill(r):
                for j in range(8):
                    bigb[r, pl.ds(16 * j, 16)] = zero16

            @pl.loop(0, ZC_PT)
            def _zero_out(t):
                cid = s + t * 16

                @pl.when(cid < ACC_R // DIVC)
                def _():
                    pltpu.sync_copy(bigb, wv_sh.at[pl.ds(cid * DIVC, DIVC)])

                @pl.when((cid >= ACC_R // DIVC) & (cid < ZC))
                def _():
                    zst = (cid - ACC_R // DIVC) * DIVC
                    pltpu.sync_copy(bigb, z_sh.at[pl.ds(zst, DIVC)])

        def scatter_pass(p):
            lo = p * RPP

            @pl.loop(0, NCH)
            def _chunk(jc):
                base = ebase + jc * CE
                dcp = pltpu.async_copy(dst_hbm.at[pl.ds(base, CE)], dst_v, sem0)
                if p == 0:
                    scp = pltpu.async_copy(src_hbm.at[pl.ds(base, CE)],
                                           src_v, sem1)
                    scp.wait()
                dcp.wait()

                @pl.loop(0, CE // 16)
                def _xf(t):
                    sl = pl.ds(t * 16, 16)
                    d = dst_v[sl]
                    rel = d - lo
                    okm = (rel >= 0) & (rel < RPP)
                    dst2_v[sl] = jnp.where(okm, rel, DUMMY)
                    zdst_v[sl] = jnp.where(okm,
                                           lax.shift_right_logical(rel, 3),
                                           Z_DUMMY)
                    slot_v[sl] = (d & 7) * 16

                if p == 0:
                    @pl.loop(0, CE // 16)
                    def _off(t):
                        sl = pl.ds(t * 16, 16)
                        srco_v[sl] = src_v[sl] + coff_n
                        dst_v[sl] = dst_v[sl] + coff_n

                    cps = [
                        pltpu.async_copy(k_hbm.at[srco_v], kb, sem0),
                        pltpu.async_copy(q_hbm.at[dst_v], qb, sem1),
                        pltpu.async_copy(v_hbm.at[srco_v], vb, sem2),
                        pltpu.async_copy(pe_hbm.at[pl.ds(coff_e + base, CE)],
                                         peb, sem3),
                    ]
                    for cp_ in cps:
                        cp_.wait()

                    @pl.loop(0, CE)
                    def _edge(i):
                        svecs = []
                        for j in range(8):
                            sl = pl.ds(16 * j, 16)
                            t = kb[i, sl] * qb[i, sl] * (peb[i, sl] * INV_SQRT_DH)
                            eob[i, sl] = t
                            svecs.append(t)
                        hs = [jnp.sum(svecs[2 * hh] + svecs[2 * hh + 1])
                              for hh in range(NHH)]
                        hv = jnp.where(masks[0], hs[0],
                                       jnp.where(masks[1], hs[1],
                                                 jnp.where(masks[2], hs[2],
                                                           hs[3])))
                        hv = jnp.exp(jnp.clip(hv, -5.0, 5.0))
                        slot = slot_v[pl.ds(i, 16)][0]
                        for j in range(8):
                            zb[i, pl.ds(16 * j, 16)] = zero16
                        zb[i, pl.ds(slot, 16)] = jnp.where(mlt4, hv, 0.0)
                        ivec = io * 0 + i
                        for hh in range(NHH):
                            bh = plsc.load_gather(zb, [ivec, io * 0 + slot + hh])
                            for j in (2 * hh, 2 * hh + 1):
                                sl = pl.ds(16 * j, 16)
                                wvb[i, sl] = vb[i, sl] * bh

                    wcps = [
                        pltpu.async_copy(eob,
                                         eo_hbm.at[pl.ds(coff_e + base, CE)],
                                         sem0),
                        pltpu.async_copy(wvb,
                                         wvs_hbm.at[pl.ds(coff_e + base, CE)],
                                         sem1),
                        pltpu.async_copy(zb,
                                         zs_hbm.at[pl.ds(coff_e + base, CE)],
                                         sem2),
                    ]
                else:
                    rcps = [
                        pltpu.async_copy(wvs_hbm.at[pl.ds(coff_e + base, CE)],
                                         wvb, sem0),
                        pltpu.async_copy(zs_hbm.at[pl.ds(coff_e + base, CE)],
                                         zb, sem1),
                    ]
                    for cp_ in rcps:
                        cp_.wait()

                pltpu.sync_copy(wvb, wv_sh.at[dst2_v], add=True)
                pltpu.sync_copy(zb, z_sh.at[zdst_v], add=True)
                if p == 0:
                    for cp_ in wcps:
                        cp_.wait()

        def divide_pass(p):
            wpc = min(RPP, N - p * RPP) // DIVC

            @pl.loop(0, -(-(RPP // DIVC) // 16))
            def _div(t):
                cid = s + t * 16

                @pl.when(cid < wpc)
                def _():
                    st = cid * DIVC
                    pltpu.sync_copy(wv_sh.at[pl.ds(st, DIVC)], bigb)
                    zst = cid * (DIVC // 8)
                    za = zst - (zst & 7)
                    off = zst & 7
                    pltpu.sync_copy(z_sh.at[pl.ds(za, 16)], zdivb)

                    @pl.loop(0, DIVC)
                    def _row(r):
                        zi = off + lax.shift_right_logical(r, 3)
                        slot = (r & 7) * 16
                        rvec = io * 0 + zi
                        for hh in range(NHH):
                            bh = plsc.load_gather(zdivb,
                                                  [rvec, io * 0 + slot + hh])
                            den = bh + 1e-6
                            for j in (2 * hh, 2 * hh + 1):
                                sl = pl.ds(16 * j, 16)
                                bigb[r, sl] = bigb[r, sl] / den

                    orow = coff_n + p * RPP + st
                    pltpu.sync_copy(bigb, ha_hbm.at[pl.ds(orow, DIVC)])

        for p in range(NPASS):
            zero_phase()
            plsc.subcore_barrier()
            scatter_pass(p)
            plsc.subcore_barrier()
            divide_pass(p)
            if p < NPASS - 1:
                plsc.subcore_barrier()

    return body


_sc_attn = _make_sc_attention()


def _edge_attention(K2, Q2, V2, pe2, src, dst, need_eout):
    eo2, ha2, _, _ = _sc_attn(K2.reshape(2 * N, 128), Q2.reshape(2 * N, 128),
                              V2.reshape(2 * N, 128), pe2.reshape(2 * E, 128),
                              src, dst)
    ha2 = ha2.reshape(2, N, 128)
    return (eo2.reshape(2, E, 128) if need_eout else None), ha2


def kernel(node_feat, edge_feat, lap_pos_enc, node_labels, edge_index, params):
    p = params
    src = edge_index[0].astype(jnp.int32)
    dst = edge_index[1].astype(jnp.int32)

    Maa = jnp.pad(p['aa_emb'] @ p['Wn'][:AA], ((0, AA - NL), (0, 0)))
    b0 = (p['bn'] + p['blap']).reshape(1, D)

    h = _node_init(node_labels, node_feat, lap_pos_enc,
                   Maa, p['Wn'][AA:], p['Wlap'], b0)
    e0, pe1 = _edge_init(edge_feat, p['We'], p['be'].reshape(1, D), p['Wpe'][0])

    pe_l = pe1
    for l in range(L):
        Q2, K2, V2 = _qkv(h, p['Wq'][l], p['Wk'][l], p['Wv'][l])
        eo2, ha2 = _edge_attention(K2, Q2, V2, pe_l, src, dst,
                                   need_eout=(l == 0))
        h = _h_update(h, ha2, p['WOh'][l], p['bOh'][l].reshape(1, D),
                      p['ln1h_g'][l].reshape(1, D), p['ln1h_b'][l].reshape(1, D),
                      p['Wf1h'][l], p['bf1h'][l].reshape(1, 2 * D),
                      p['Wf2h'][l], p['bf2h'][l].reshape(1, D),
                      p['ln2h_g'][l].reshape(1, D), p['ln2h_b'][l].reshape(1, D))
        if l == 0:
            pe_l = _e_update(e0, eo2, p['WOe'][l], p['bOe'][l].reshape(1, D),
                             p['ln1e_g'][l].reshape(1, D), p['ln1e_b'][l].reshape(1, D),
                             p['Wf1e'][l], p['bf1e'][l].reshape(1, 2 * D),
                             p['Wf2e'][l], p['bf2e'][l].reshape(1, D),
                             p['ln2e_g'][l].reshape(1, D), p['ln2e_b'][l].reshape(1, D),
                             p['Wpe'][1])

    R2Wp = jnp.pad(p['R2W'], ((0, 0), (0, 16 - NC)))
    R2bp = jnp.pad(p['R2b'], (0, 16 - NC)).reshape(1, 16)
    return _readout(h, p['R0W'], p['R0b'].reshape(1, D // 2),
                    p['R1W'], p['R1b'].reshape(1, D // 4), R2Wp, R2bp)
